# baseline (speedup 1.0000x reference)
import functools

import jax
import jax.numpy as jnp
from jax import lax
from jax.experimental import pallas as pl
from jax.experimental.pallas import tpu as pltpu
from jax.experimental.pallas import tpu_sc as plsc

_N, _E, _D, _H = 10000, 320000, 128, 128
_B = 64
_EPS_BN = 1e-3

_NC, _NS = 2, 16
_NW = _NC * _NS
_CH = 125
_ROWS = _E // _CH
_RPW = _ROWS // _NW
_G = 40
_NPA = 624
_ZCH = 104
_ZB = 48
_TAIL = _N - _NS * _NPA

_BR = 2000
_NBLK = _N // _BR


def _agg_body(h_hbm, src_hbm, dst_hbm, p0_hbm, p1_hbm,
              sidx, didx, buf0, buf1, zbuf, acc, sem0, sem1, semz):
    c = lax.axis_index("c")
    s = lax.axis_index("s")
    wid = s * _NC + c
    row0 = wid * _RPW

    def _fire(j, buf, sem):
        return pltpu.async_copy(h_hbm.at[sidx.at[j]], buf, sem)

    def _wait(buf, sem):
        pltpu.make_async_copy(h_hbm.at[sidx.at[0]], buf, sem).wait()

    pltpu.sync_copy(src_hbm.at[pl.ds(row0, _G)], sidx)
    pltpu.sync_copy(dst_hbm.at[pl.ds(row0, _G)], didx)
    _fire(0, buf0, sem0)
    _fire(1, buf1, sem1)

    zbase = s * _NPA

    @pl.when(c == 0)
    def _():
        for k in range(_NPA // _ZCH):
            pltpu.async_copy(h_hbm.at[pl.ds(zbase + k * _ZCH, _ZCH)],
                             acc.at[pl.ds(zbase + k * _ZCH, _ZCH)], semz)

        @pl.when(s == _NS - 1)
        def _():
            pltpu.async_copy(h_hbm.at[pl.ds(_NS * _NPA, _TAIL)],
                             acc.at[pl.ds(_NS * _NPA, _TAIL)], semz)
        for k in range(_NPA // _ZCH):
            pltpu.make_async_copy(h_hbm.at[pl.ds(zbase + k * _ZCH, _ZCH)],
                                  acc.at[pl.ds(zbase + k * _ZCH, _ZCH)], semz).wait()

        @pl.when(s == _NS - 1)
        def _():
            pltpu.make_async_copy(h_hbm.at[pl.ds(_NS * _NPA, _TAIL)],
                                  acc.at[pl.ds(_NS * _NPA, _TAIL)], semz).wait()

    @pl.when(c == 1)
    def _():
        def _zrow(i, carry):
            for j in range(_H // 16):
                zbuf[i, pl.ds(j * 16, 16)] = jnp.zeros((16,), jnp.float32)
            return carry
        lax.fori_loop(0, _ZB, _zrow, 0)
        for k in range(_NPA // _ZB):
            pltpu.async_copy(zbuf, acc.at[pl.ds(zbase + k * _ZB, _ZB)], semz)

        @pl.when(s == _NS - 1)
        def _():
            pltpu.async_copy(zbuf.at[pl.ds(0, _TAIL)],
                             acc.at[pl.ds(_NS * _NPA, _TAIL)], semz)
        for k in range(_NPA // _ZB):
            pltpu.make_async_copy(zbuf, acc.at[pl.ds(zbase + k * _ZB, _ZB)],
                                  semz).wait()

        @pl.when(s == _NS - 1)
        def _():
            pltpu.make_async_copy(zbuf.at[pl.ds(0, _TAIL)],
                                  acc.at[pl.ds(_NS * _NPA, _TAIL)], semz).wait()
    plsc.subcore_barrier()

    for g in range(_RPW // _G):
        if g > 0:
            pltpu.sync_copy(src_hbm.at[pl.ds(row0 + g * _G, _G)], sidx)
            pltpu.sync_copy(dst_hbm.at[pl.ds(row0 + g * _G, _G)], didx)
            _fire(0, buf0, sem0)
            _fire(1, buf1, sem1)

        def _pair(t, carry):
            j = t * 2
            _wait(buf0, sem0)
            pltpu.sync_copy(buf0, acc.at[didx.at[j]], add=True)

            @pl.when(j + 2 < _G)
            def _():
                _fire(j + 2, buf0, sem0)

            _wait(buf1, sem1)
            pltpu.sync_copy(buf1, acc.at[didx.at[j + 1]], add=True)

            @pl.when(j + 3 < _G)
            def _():
                _fire(j + 3, buf1, sem1)
            return carry

        lax.fori_loop(0, _G // 2, _pair, 0)
    plsc.subcore_barrier()

    obase = s * _NPA

    @pl.when(c == 0)
    def _():
        pltpu.sync_copy(acc.at[pl.ds(obase, _NPA)], p0_hbm.at[pl.ds(obase, _NPA)])

        @pl.when(s == _NS - 1)
        def _():
            pltpu.sync_copy(acc.at[pl.ds(_NS * _NPA, _TAIL)],
                            p0_hbm.at[pl.ds(_NS * _NPA, _TAIL)])

    @pl.when(c == 1)
    def _():
        pltpu.sync_copy(acc.at[pl.ds(obase, _NPA)], p1_hbm.at[pl.ds(obase, _NPA)])

        @pl.when(s == _NS - 1)
        def _():
            pltpu.sync_copy(acc.at[pl.ds(_NS * _NPA, _TAIL)],
                            p1_hbm.at[pl.ds(_NS * _NPA, _TAIL)])


_agg = functools.partial(
    pl.kernel,
    mesh=plsc.VectorSubcoreMesh(core_axis_name="c", subcore_axis_name="s"),
    out_type=(jax.ShapeDtypeStruct((_N, _H), jnp.float32),
              jax.ShapeDtypeStruct((_N, _H), jnp.float32)),
    scratch_types=[
        pltpu.VMEM((_G, _CH), jnp.int32),
        pltpu.VMEM((_G, _CH), jnp.int32),
        pltpu.VMEM((_CH, _H), jnp.float32),
        pltpu.VMEM((_CH, _H), jnp.float32),
        pltpu.VMEM((_ZB, _H), jnp.float32),
        pltpu.VMEM_SHARED((_N, _H), jnp.float32),
        pltpu.SemaphoreType.DMA,
        pltpu.SemaphoreType.DMA,
        pltpu.SemaphoreType.DMA,
    ],
)(_agg_body)


def _stage1_body(x_ref, W1_ref, b1_ref, g1_ref, be1_ref, m1_ref, v1_ref,
                 W2_ref, b2_ref, out_ref):
    h = jnp.dot(x_ref[...], W1_ref[...], preferred_element_type=jnp.float32)
    h = jnp.maximum(h + b1_ref[...], 0.0)
    h = (h - m1_ref[...]) / jnp.sqrt(v1_ref[...] + _EPS_BN) * g1_ref[...] + be1_ref[...]
    h = jnp.dot(h, W2_ref[...], preferred_element_type=jnp.float32)
    out_ref[...] = jnp.maximum(h + b2_ref[...], 0.0)


def _stage2_body(p0_ref, p1_ref, W3_ref, b3_ref, g2_ref, be2_ref,
                 m2_ref, v2_ref, out_ref):
    u = p0_ref[...] + p1_ref[...]
    h = jnp.dot(u, W3_ref[...], preferred_element_type=jnp.float32)
    h = jnp.maximum(h + b3_ref[...], 0.0)
    out_ref[...] = (h - m2_ref[...]) / jnp.sqrt(v2_ref[...] + _EPS_BN) * g2_ref[...] + be2_ref[...]


def _stage3_body(q0_ref, q1_ref, W4_ref, b4_ref, batch_ref, out_ref):
    i = pl.program_id(0)
    u = q0_ref[...] + q1_ref[...]
    h = jnp.dot(u, W4_ref[...], preferred_element_type=jnp.float32)
    h = jnp.maximum(h + b4_ref[...], 0.0)
    b = batch_ref[0, 0, :]
    onehot = (b[:, None] == lax.broadcasted_iota(jnp.int32, (_BR, _B), 1))
    contrib = lax.dot_general(onehot.astype(jnp.float32), h,
                              (((0,), (0,)), ((), ())),
                              preferred_element_type=jnp.float32)

    @pl.when(i == 0)
    def _():
        out_ref[...] = jnp.zeros_like(out_ref)

    out_ref[...] += contrib


def _vec_spec():
    return pl.BlockSpec((1, _H), lambda i: (0, 0))


def _row_spec():
    return pl.BlockSpec((_BR, _H), lambda i: (i, 0))


def _w_spec():
    return pl.BlockSpec((_H, _H), lambda i: (0, 0))


def _stage1(x, W1, b1, g1, be1, m1, v1, W2, b2):
    return pl.pallas_call(
        _stage1_body,
        grid=(_NBLK,),
        in_specs=[_row_spec(), _w_spec(), _vec_spec(), _vec_spec(), _vec_spec(),
                  _vec_spec(), _vec_spec(), _w_spec(), _vec_spec()],
        out_specs=_row_spec(),
        out_shape=jax.ShapeDtypeStruct((_N, _H), jnp.float32),
    )(x, W1, b1, g1, be1, m1, v1, W2, b2)


def _stage2(p0, p1, W3, b3, g2, be2, m2, v2):
    return pl.pallas_call(
        _stage2_body,
        grid=(_NBLK,),
        in_specs=[_row_spec(), _row_spec(), _w_spec(), _vec_spec(),
                  _vec_spec(), _vec_spec(), _vec_spec(), _vec_spec()],
        out_specs=_row_spec(),
        out_shape=jax.ShapeDtypeStruct((_N, _H), jnp.float32),
    )(p0, p1, W3, b3, g2, be2, m2, v2)


def _stage3(q0, q1, W4, b4, batch3):
    return pl.pallas_call(
        _stage3_body,
        grid=(_NBLK,),
        in_specs=[_row_spec(), _row_spec(), _w_spec(), _vec_spec(),
                  pl.BlockSpec((1, 1, _BR), lambda i: (i, 0, 0))],
        out_specs=pl.BlockSpec((_B, _H), lambda i: (0, 0)),
        out_shape=jax.ShapeDtypeStruct((_B, _H), jnp.float32),
    )(q0, q1, W4, b4, batch3)


def kernel(x, edge_index, batch, W1, b1, g1, be1, m1, v1, W2, b2, W3, b3,
           g2, be2, m2, v2, W4, b4):
    src2 = edge_index[:, 0:1].reshape(_ROWS, _CH)
    dst2 = edge_index[:, 1:2].reshape(_ROWS, _CH)
    batch3 = batch.reshape(_NBLK, 1, _BR)
    r = lambda v: v.reshape(1, _H)

    h2 = _stage1(x, W1, r(b1), r(g1), r(be1), r(m1), r(v1), W2, r(b2))
    p0, p1 = _agg(h2, src2, dst2)
    h3 = _stage2(p0, p1, W3, r(b3), r(g2), r(be2), r(m2), r(v2))
    q0, q1 = _agg(h3, src2, dst2)
    return _stage3(q0, q1, W4, r(b4), batch3)


# speedup vs baseline: 1.0026x; 1.0026x over previous
import functools

import jax
import jax.numpy as jnp
from jax import lax
from jax.experimental import pallas as pl
from jax.experimental.pallas import tpu as pltpu
from jax.experimental.pallas import tpu_sc as plsc

_N, _E, _D, _H = 10000, 320000, 128, 128
_B = 64
_EPS_BN = 1e-3

_NC, _NS = 2, 16
_NW = _NC * _NS
_CH = 125
_ROWS = _E // _CH
_RPW = _ROWS // _NW
_G = 40
_NPA = 624
_ZCH = 104
_ZB = 48
_TAIL = _N - _NS * _NPA

_BR = 2000
_NBLK = _N // _BR


def _agg_body(h_hbm, src_hbm, dst_hbm, p0_hbm, p1_hbm,
              sidx, didx, buf0, buf1, zbuf, acc, sem0, sem1, semz):
    c = lax.axis_index("c")
    s = lax.axis_index("s")
    wid = s * _NC + c
    row0 = wid * _RPW

    def _fire(j, buf, sem):
        return pltpu.async_copy(h_hbm.at[sidx.at[j]], buf, sem)

    def _wait(buf, sem):
        pltpu.make_async_copy(h_hbm.at[sidx.at[0]], buf, sem).wait()

    pltpu.sync_copy(src_hbm.at[pl.ds(row0, _G)], sidx)
    pltpu.sync_copy(dst_hbm.at[pl.ds(row0, _G)], didx)
    _fire(0, buf0, sem0)
    _fire(1, buf1, sem1)

    zbase = s * _NPA

    @pl.when(c == 0)
    def _():
        for k in range(_NPA // _ZCH):
            pltpu.async_copy(h_hbm.at[pl.ds(zbase + k * _ZCH, _ZCH)],
                             acc.at[pl.ds(zbase + k * _ZCH, _ZCH)], semz)

        @pl.when(s == _NS - 1)
        def _():
            pltpu.async_copy(h_hbm.at[pl.ds(_NS * _NPA, _TAIL)],
                             acc.at[pl.ds(_NS * _NPA, _TAIL)], semz)
        for k in range(_NPA // _ZCH):
            pltpu.make_async_copy(h_hbm.at[pl.ds(zbase + k * _ZCH, _ZCH)],
                                  acc.at[pl.ds(zbase + k * _ZCH, _ZCH)], semz).wait()

        @pl.when(s == _NS - 1)
        def _():
            pltpu.make_async_copy(h_hbm.at[pl.ds(_NS * _NPA, _TAIL)],
                                  acc.at[pl.ds(_NS * _NPA, _TAIL)], semz).wait()

    @pl.when(c == 1)
    def _():
        def _zrow(i, carry):
            for j in range(_H // 16):
                zbuf[i, pl.ds(j * 16, 16)] = jnp.zeros((16,), jnp.float32)
            return carry
        lax.fori_loop(0, _ZB, _zrow, 0)
        for k in range(_NPA // _ZB):
            pltpu.async_copy(zbuf, acc.at[pl.ds(zbase + k * _ZB, _ZB)], semz)

        @pl.when(s == _NS - 1)
        def _():
            pltpu.async_copy(zbuf.at[pl.ds(0, _TAIL)],
                             acc.at[pl.ds(_NS * _NPA, _TAIL)], semz)
        for k in range(_NPA // _ZB):
            pltpu.make_async_copy(zbuf, acc.at[pl.ds(zbase + k * _ZB, _ZB)],
                                  semz).wait()

        @pl.when(s == _NS - 1)
        def _():
            pltpu.make_async_copy(zbuf.at[pl.ds(0, _TAIL)],
                                  acc.at[pl.ds(_NS * _NPA, _TAIL)], semz).wait()
    plsc.subcore_barrier()

    for g in range(_RPW // _G):
        if g > 0:
            pltpu.sync_copy(src_hbm.at[pl.ds(row0 + g * _G, _G)], sidx)
            pltpu.sync_copy(dst_hbm.at[pl.ds(row0 + g * _G, _G)], didx)
            _fire(0, buf0, sem0)
            _fire(1, buf1, sem1)

        def _pair(t, carry):
            j = t * 2
            _wait(buf0, sem0)
            pltpu.sync_copy(buf0, acc.at[didx.at[j]], add=True)

            @pl.when(j + 2 < _G)
            def _():
                _fire(j + 2, buf0, sem0)

            _wait(buf1, sem1)
            pltpu.sync_copy(buf1, acc.at[didx.at[j + 1]], add=True)

            @pl.when(j + 3 < _G)
            def _():
                _fire(j + 3, buf1, sem1)
            return carry

        lax.fori_loop(0, _G // 2, _pair, 0)
    plsc.subcore_barrier()

    obase = s * _NPA

    @pl.when(c == 0)
    def _():
        pltpu.sync_copy(acc.at[pl.ds(obase, _NPA)], p0_hbm.at[pl.ds(obase, _NPA)])

        @pl.when(s == _NS - 1)
        def _():
            pltpu.sync_copy(acc.at[pl.ds(_NS * _NPA, _TAIL)],
                            p0_hbm.at[pl.ds(_NS * _NPA, _TAIL)])

    @pl.when(c == 1)
    def _():
        pltpu.sync_copy(acc.at[pl.ds(obase, _NPA)], p1_hbm.at[pl.ds(obase, _NPA)])

        @pl.when(s == _NS - 1)
        def _():
            pltpu.sync_copy(acc.at[pl.ds(_NS * _NPA, _TAIL)],
                            p1_hbm.at[pl.ds(_NS * _NPA, _TAIL)])


_agg = functools.partial(
    pl.kernel,
    mesh=plsc.VectorSubcoreMesh(core_axis_name="c", subcore_axis_name="s"),
    compiler_params=pltpu.CompilerParams(use_tc_tiling_on_sc=False),
    out_type=(jax.ShapeDtypeStruct((_N, _H), jnp.float32),
              jax.ShapeDtypeStruct((_N, _H), jnp.float32)),
    scratch_types=[
        pltpu.VMEM((_G, _CH), jnp.int32),
        pltpu.VMEM((_G, _CH), jnp.int32),
        pltpu.VMEM((_CH, _H), jnp.float32),
        pltpu.VMEM((_CH, _H), jnp.float32),
        pltpu.VMEM((_ZB, _H), jnp.float32),
        pltpu.VMEM_SHARED((_N, _H), jnp.float32),
        pltpu.SemaphoreType.DMA,
        pltpu.SemaphoreType.DMA,
        pltpu.SemaphoreType.DMA,
    ],
)(_agg_body)


def _stage1_body(x_ref, W1_ref, b1_ref, g1_ref, be1_ref, m1_ref, v1_ref,
                 W2_ref, b2_ref, out_ref):
    h = jnp.dot(x_ref[...], W1_ref[...], preferred_element_type=jnp.float32)
    h = jnp.maximum(h + b1_ref[...], 0.0)
    h = (h - m1_ref[...]) / jnp.sqrt(v1_ref[...] + _EPS_BN) * g1_ref[...] + be1_ref[...]
    h = jnp.dot(h, W2_ref[...], preferred_element_type=jnp.float32)
    out_ref[...] = jnp.maximum(h + b2_ref[...], 0.0)


def _stage2_body(p0_ref, p1_ref, W3_ref, b3_ref, g2_ref, be2_ref,
                 m2_ref, v2_ref, out_ref):
    u = p0_ref[...] + p1_ref[...]
    h = jnp.dot(u, W3_ref[...], preferred_element_type=jnp.float32)
    h = jnp.maximum(h + b3_ref[...], 0.0)
    out_ref[...] = (h - m2_ref[...]) / jnp.sqrt(v2_ref[...] + _EPS_BN) * g2_ref[...] + be2_ref[...]


def _stage3_body(q0_ref, q1_ref, W4_ref, b4_ref, batch_ref, out_ref):
    i = pl.program_id(0)
    u = q0_ref[...] + q1_ref[...]
    h = jnp.dot(u, W4_ref[...], preferred_element_type=jnp.float32)
    h = jnp.maximum(h + b4_ref[...], 0.0)
    b = batch_ref[0, 0, :]
    onehot = (b[:, None] == lax.broadcasted_iota(jnp.int32, (_BR, _B), 1))
    contrib = lax.dot_general(onehot.astype(jnp.float32), h,
                              (((0,), (0,)), ((), ())),
                              preferred_element_type=jnp.float32)

    @pl.when(i == 0)
    def _():
        out_ref[...] = jnp.zeros_like(out_ref)

    out_ref[...] += contrib


def _vec_spec():
    return pl.BlockSpec((1, _H), lambda i: (0, 0))


def _row_spec():
    return pl.BlockSpec((_BR, _H), lambda i: (i, 0))


def _w_spec():
    return pl.BlockSpec((_H, _H), lambda i: (0, 0))


def _stage1(x, W1, b1, g1, be1, m1, v1, W2, b2):
    return pl.pallas_call(
        _stage1_body,
        grid=(_NBLK,),
        in_specs=[_row_spec(), _w_spec(), _vec_spec(), _vec_spec(), _vec_spec(),
                  _vec_spec(), _vec_spec(), _w_spec(), _vec_spec()],
        out_specs=_row_spec(),
        out_shape=jax.ShapeDtypeStruct((_N, _H), jnp.float32),
    )(x, W1, b1, g1, be1, m1, v1, W2, b2)


def _stage2(p0, p1, W3, b3, g2, be2, m2, v2):
    return pl.pallas_call(
        _stage2_body,
        grid=(_NBLK,),
        in_specs=[_row_spec(), _row_spec(), _w_spec(), _vec_spec(),
                  _vec_spec(), _vec_spec(), _vec_spec(), _vec_spec()],
        out_specs=_row_spec(),
        out_shape=jax.ShapeDtypeStruct((_N, _H), jnp.float32),
    )(p0, p1, W3, b3, g2, be2, m2, v2)


def _stage3(q0, q1, W4, b4, batch3):
    return pl.pallas_call(
        _stage3_body,
        grid=(_NBLK,),
        in_specs=[_row_spec(), _row_spec(), _w_spec(), _vec_spec(),
                  pl.BlockSpec((1, 1, _BR), lambda i: (i, 0, 0))],
        out_specs=pl.BlockSpec((_B, _H), lambda i: (0, 0)),
        out_shape=jax.ShapeDtypeStruct((_B, _H), jnp.float32),
    )(q0, q1, W4, b4, batch3)


def kernel(x, edge_index, batch, W1, b1, g1, be1, m1, v1, W2, b2, W3, b3,
           g2, be2, m2, v2, W4, b4):
    src2 = edge_index[:, 0:1].reshape(_ROWS, _CH)
    dst2 = edge_index[:, 1:2].reshape(_ROWS, _CH)
    batch3 = batch.reshape(_NBLK, 1, _BR)
    r = lambda v: v.reshape(1, _H)

    h2 = _stage1(x, W1, r(b1), r(g1), r(be1), r(m1), r(v1), W2, r(b2))
    p0, p1 = _agg(h2, src2, dst2)
    h3 = _stage2(p0, p1, W3, r(b3), r(g2), r(be2), r(m2), r(v2))
    q0, q1 = _agg(h3, src2, dst2)
    return _stage3(q0, q1, W4, r(b4), batch3)


# speedup vs baseline: 1.0047x; 1.0021x over previous
import functools

import jax
import jax.numpy as jnp
from jax import lax
from jax.experimental import pallas as pl
from jax.experimental.pallas import tpu as pltpu
from jax.experimental.pallas import tpu_sc as plsc

_N, _E, _D, _H = 10000, 320000, 128, 128
_B = 64
_EPS_BN = 1e-3

_NC, _NS = 2, 16
_NW = _NC * _NS
_CH = 125
_ROWS = _E // _CH
_RPW = _ROWS // _NW
_G = 40
_NPA = 624
_ZCH = 104
_ZB = 48
_TAIL = _N - _NS * _NPA

_BR = 2000
_NBLK = _N // _BR


def _agg_body(h_hbm, src_hbm, dst_hbm, p0_hbm, p1_hbm,
              sidx, didx, buf0, buf1, zbuf, acc, sem0, sem1, semz):
    c = lax.axis_index("c")
    s = lax.axis_index("s")
    wid = s * _NC + c
    row0 = wid * _RPW

    def _fire(j, buf, sem):
        return pltpu.async_copy(h_hbm.at[sidx.at[j]], buf, sem)

    def _wait(buf, sem):
        pltpu.make_async_copy(h_hbm.at[sidx.at[0]], buf, sem).wait()

    pltpu.sync_copy(src_hbm.at[pl.ds(row0, _G)], sidx)
    pltpu.sync_copy(dst_hbm.at[pl.ds(row0, _G)], didx)
    _fire(0, buf0, sem0)
    _fire(1, buf1, sem1)

    zbase = s * _NPA

    @pl.when(c == 0)
    def _():
        for k in range(_NPA // _ZCH):
            pltpu.async_copy(h_hbm.at[pl.ds(zbase + k * _ZCH, _ZCH)],
                             acc.at[pl.ds(zbase + k * _ZCH, _ZCH)], semz)

        @pl.when(s == _NS - 1)
        def _():
            pltpu.async_copy(h_hbm.at[pl.ds(_NS * _NPA, _TAIL)],
                             acc.at[pl.ds(_NS * _NPA, _TAIL)], semz)
        for k in range(_NPA // _ZCH):
            pltpu.make_async_copy(h_hbm.at[pl.ds(zbase + k * _ZCH, _ZCH)],
                                  acc.at[pl.ds(zbase + k * _ZCH, _ZCH)], semz).wait()

        @pl.when(s == _NS - 1)
        def _():
            pltpu.make_async_copy(h_hbm.at[pl.ds(_NS * _NPA, _TAIL)],
                                  acc.at[pl.ds(_NS * _NPA, _TAIL)], semz).wait()

    @pl.when(c == 1)
    def _():
        def _zrow(i, carry):
            for j in range(_H // 16):
                zbuf[i, pl.ds(j * 16, 16)] = jnp.zeros((16,), jnp.float32)
            return carry
        lax.fori_loop(0, _ZB, _zrow, 0)
        for k in range(_NPA // _ZB):
            pltpu.async_copy(zbuf, acc.at[pl.ds(zbase + k * _ZB, _ZB)], semz)

        @pl.when(s == _NS - 1)
        def _():
            pltpu.async_copy(zbuf.at[pl.ds(0, _TAIL)],
                             acc.at[pl.ds(_NS * _NPA, _TAIL)], semz)
        for k in range(_NPA // _ZB):
            pltpu.make_async_copy(zbuf, acc.at[pl.ds(zbase + k * _ZB, _ZB)],
                                  semz).wait()

        @pl.when(s == _NS - 1)
        def _():
            pltpu.make_async_copy(zbuf.at[pl.ds(0, _TAIL)],
                                  acc.at[pl.ds(_NS * _NPA, _TAIL)], semz).wait()
    plsc.subcore_barrier()

    for g in range(_RPW // _G):
        if g > 0:
            pltpu.sync_copy(src_hbm.at[pl.ds(row0 + g * _G, _G)], sidx)
            pltpu.sync_copy(dst_hbm.at[pl.ds(row0 + g * _G, _G)], didx)
            _fire(0, buf0, sem0)
            _fire(1, buf1, sem1)

        def _pair(t, carry):
            j = t * 2
            _wait(buf0, sem0)
            pltpu.sync_copy(buf0, acc.at[didx.at[j]], add=True)

            @pl.when(j + 2 < _G)
            def _():
                _fire(j + 2, buf0, sem0)

            _wait(buf1, sem1)
            pltpu.sync_copy(buf1, acc.at[didx.at[j + 1]], add=True)

            @pl.when(j + 3 < _G)
            def _():
                _fire(j + 3, buf1, sem1)
            return carry

        lax.fori_loop(0, _G // 2, _pair, 0)
    plsc.subcore_barrier()

    obase = s * _NPA

    @pl.when(c == 0)
    def _():
        pltpu.sync_copy(acc.at[pl.ds(obase, _NPA)], p0_hbm.at[pl.ds(obase, _NPA)])

        @pl.when(s == _NS - 1)
        def _():
            pltpu.sync_copy(acc.at[pl.ds(_NS * _NPA, _TAIL)],
                            p0_hbm.at[pl.ds(_NS * _NPA, _TAIL)])

    @pl.when(c == 1)
    def _():
        pltpu.sync_copy(acc.at[pl.ds(obase, _NPA)], p1_hbm.at[pl.ds(obase, _NPA)])

        @pl.when(s == _NS - 1)
        def _():
            pltpu.sync_copy(acc.at[pl.ds(_NS * _NPA, _TAIL)],
                            p1_hbm.at[pl.ds(_NS * _NPA, _TAIL)])


_agg = functools.partial(
    pl.kernel,
    mesh=plsc.VectorSubcoreMesh(core_axis_name="c", subcore_axis_name="s"),
    out_type=(jax.ShapeDtypeStruct((_N, _H), jnp.float32),
              jax.ShapeDtypeStruct((_N, _H), jnp.float32)),
    scratch_types=[
        pltpu.VMEM((_G, _CH), jnp.int32),
        pltpu.VMEM((_G, _CH), jnp.int32),
        pltpu.VMEM((_CH, _H), jnp.float32),
        pltpu.VMEM((_CH, _H), jnp.float32),
        pltpu.VMEM((_ZB, _H), jnp.float32),
        pltpu.VMEM_SHARED((_N, _H), jnp.float32),
        pltpu.SemaphoreType.DMA,
        pltpu.SemaphoreType.DMA,
        pltpu.SemaphoreType.DMA,
    ],
)(_agg_body)


def _stage1_body(x_ref, W1_ref, b1_ref, g1_ref, be1_ref, m1_ref, v1_ref,
                 W2_ref, b2_ref, out_ref):
    h = jnp.dot(x_ref[...], W1_ref[...], preferred_element_type=jnp.float32)
    h = jnp.maximum(h + b1_ref[...], 0.0)
    h = (h - m1_ref[...]) / jnp.sqrt(v1_ref[...] + _EPS_BN) * g1_ref[...] + be1_ref[...]
    h = jnp.dot(h, W2_ref[...], preferred_element_type=jnp.float32)
    out_ref[...] = jnp.maximum(h + b2_ref[...], 0.0)


def _stage2_body(p0_ref, p1_ref, W3_ref, b3_ref, g2_ref, be2_ref,
                 m2_ref, v2_ref, out_ref):
    u = p0_ref[...] + p1_ref[...]
    h = jnp.dot(u, W3_ref[...], preferred_element_type=jnp.float32)
    h = jnp.maximum(h + b3_ref[...], 0.0)
    out_ref[...] = (h - m2_ref[...]) / jnp.sqrt(v2_ref[...] + _EPS_BN) * g2_ref[...] + be2_ref[...]


def _stage3_body(q0_ref, q1_ref, W4_ref, b4_ref, batch_ref, out_ref):
    i = pl.program_id(0)
    u = q0_ref[...] + q1_ref[...]
    h = jnp.dot(u, W4_ref[...], preferred_element_type=jnp.float32)
    h = jnp.maximum(h + b4_ref[...], 0.0)
    b = batch_ref[0, 0, :]
    onehot = (b[:, None] == lax.broadcasted_iota(jnp.int32, (_BR, _B), 1))
    contrib = lax.dot_general(onehot.astype(jnp.float32), h,
                              (((0,), (0,)), ((), ())),
                              preferred_element_type=jnp.float32)

    @pl.when(i == 0)
    def _():
        out_ref[...] = jnp.zeros_like(out_ref)

    out_ref[...] += contrib


def _vec_spec():
    return pl.BlockSpec((1, _H), lambda i: (0, 0))


def _row_spec():
    return pl.BlockSpec((_BR, _H), lambda i: (i, 0))


def _w_spec():
    return pl.BlockSpec((_H, _H), lambda i: (0, 0))


def _stage1(x, W1, b1, g1, be1, m1, v1, W2, b2):
    return pl.pallas_call(
        _stage1_body,
        grid=(_NBLK,),
        in_specs=[_row_spec(), _w_spec(), _vec_spec(), _vec_spec(), _vec_spec(),
                  _vec_spec(), _vec_spec(), _w_spec(), _vec_spec()],
        out_specs=_row_spec(),
        out_shape=jax.ShapeDtypeStruct((_N, _H), jnp.float32),
    )(x, W1, b1, g1, be1, m1, v1, W2, b2)


def _stage2(p0, p1, W3, b3, g2, be2, m2, v2):
    return pl.pallas_call(
        _stage2_body,
        grid=(_NBLK,),
        in_specs=[_row_spec(), _row_spec(), _w_spec(), _vec_spec(),
                  _vec_spec(), _vec_spec(), _vec_spec(), _vec_spec()],
        out_specs=_row_spec(),
        out_shape=jax.ShapeDtypeStruct((_N, _H), jnp.float32),
    )(p0, p1, W3, b3, g2, be2, m2, v2)


def _stage3(q0, q1, W4, b4, batch3):
    return pl.pallas_call(
        _stage3_body,
        grid=(_NBLK,),
        in_specs=[_row_spec(), _row_spec(), _w_spec(), _vec_spec(),
                  pl.BlockSpec((1, 1, _BR), lambda i: (i, 0, 0))],
        out_specs=pl.BlockSpec((_B, _H), lambda i: (0, 0)),
        out_shape=jax.ShapeDtypeStruct((_B, _H), jnp.float32),
    )(q0, q1, W4, b4, batch3)


def kernel(x, edge_index, batch, W1, b1, g1, be1, m1, v1, W2, b2, W3, b3,
           g2, be2, m2, v2, W4, b4):
    src2 = edge_index[:, 0:1].reshape(_ROWS, _CH)
    dst2 = edge_index[:, 1:2].reshape(_ROWS, _CH)
    batch3 = batch.reshape(_NBLK, 1, _BR)
    r = lambda v: v.reshape(1, _H)

    h2 = _stage1(x, W1, r(b1), r(g1), r(be1), r(m1), r(v1), W2, r(b2))
    p0, p1 = _agg(h2, src2, dst2)
    h3 = _stage2(p0, p1, W3, r(b3), r(g2), r(be2), r(m2), r(v2))
    q0, q1 = _agg(h3, src2, dst2)
    return _stage3(q0, q1, W4, r(b4), batch3)
